# baseline (speedup 1.0000x reference)
import dataclasses
import functools

import jax
import jax.numpy as jnp
from jax import lax
from jax.experimental import pallas as pl
from jax.experimental.pallas import tpu as pltpu
from jax.experimental.pallas import tpu_sc as plsc

_NW, _NT, _ND = 50000, 5000, 10000
_D = 128
_CC = 32
_NCH = _D // _CC
_B = 256
_G = 256
_SSB = 4
_NTILES = 16

_NWP = 50176
_TT0, _WD0, _TD0 = 5120, 10240, 20480
_NRP = 30720
_NTP = 5120
_NDP = 10240

_EMULT = _NTILES * _B * _SSB


def _pack_edges(src, dst, w, trim_lo, trim_n):
    ne = src.shape[0]
    ne_pad = ((ne + _EMULT - 1) // _EMULT) * _EMULT
    pad = ne_pad - ne
    pdst = trim_lo + jnp.arange(pad, dtype=jnp.int32) % trim_n
    wb = lax.bitcast_convert_type(w, jnp.int32)
    spt = ne_pad // _NTILES
    p_t = [pad // _NTILES + (1 if t < pad % _NTILES else 0)
           for t in range(_NTILES)]
    r_t = [spt - p for p in p_t]
    r_off = [sum(r_t[:t]) for t in range(_NTILES + 1)]
    p_off = [sum(p_t[:t]) for t in range(_NTILES + 1)]

    def spread(real, padv):
        parts = []
        for t in range(_NTILES):
            parts.append(real[r_off[t]:r_off[t + 1]])
            parts.append(padv[p_off[t]:p_off[t + 1]])
        return jnp.concatenate(parts)

    src = spread(src, jnp.zeros((pad,), jnp.int32))
    dst = spread(dst, pdst)
    wb = spread(wb, jnp.zeros((pad,), jnp.int32))
    return jnp.stack([src.reshape(-1, _B), dst.reshape(-1, _B),
                      wb.reshape(-1, _B)], axis=1)


def _sc_aggregate(gc, e3w, e3r):
    mesh = plsc.VectorSubcoreMesh(core_axis_name="c", subcore_axis_name="s")
    f32 = jnp.float32
    out_type = ([jax.ShapeDtypeStruct((_NWP, _CC), f32) for _ in range(5)]
                + [jax.ShapeDtypeStruct((_NRP, _CC), f32) for _ in range(5)])
    scratch_types = [
        pltpu.VMEM((_G, _CC), f32),
        pltpu.VMEM((_G, _CC), f32),
        pltpu.VMEM((_SSB, 3, _B), jnp.int32),
        pltpu.VMEM_SHARED((_NWP, _CC), f32),
        pltpu.SemaphoreType.DMA((2,)),
        pltpu.SemaphoreType.DMA((2,)),
    ]
    cparams = pltpu.CompilerParams(use_tc_tiling_on_sc=False)
    if "needs_layout_passes" in pltpu.CompilerParams.__dataclass_fields__:
        cparams = dataclasses.replace(cparams, needs_layout_passes=False)

    @functools.partial(pl.kernel, out_type=out_type, mesh=mesh,
                       scratch_types=scratch_types,
                       compiler_params=cparams)
    def agg(g0, g1, g2, g3, e3w_h, e3r_h,
            Sw0, Sw1, Sw2, Sw3, Pw, Sr0, Sr1, Sr2, Sr3, Pr,
            rbA, rbB, est, acc, gsem, ssem):
        core = lax.axis_index("c")
        tid = lax.axis_index("s")
        rb = [rbA, rbB]

        lane = lax.broadcasted_iota(jnp.int32, (16,), 0)
        e0f = jnp.where(lane == 0, 1.0, 0.0)
        e1f = jnp.where(lane == 1, 1.0, 0.0)
        zeros16 = jnp.zeros((16,), f32)

        def one_pass(G, e3ref, out_ref, n_rows, pairs_mode):
            nblk = e3ref.shape[0] // _NTILES
            nsuper = nblk // _SSB
            srow = tid * nblk

            def drain_scatters(x):
                pltpu.make_async_copy(rb[x], acc.at[pl.ds(0, _G)],
                                      ssem.at[x]).wait()

            def drain_gathers(x):
                pltpu.make_async_copy(G.at[pl.ds(0, _G)], rb[x],
                                      gsem.at[x]).wait()

            def fire_gathers(x, jj):
                pltpu.async_copy(G.at[est.at[jj, 0]], rb[x], gsem.at[x])

            def fire_scatters(x, jj):
                pltpu.async_copy(rb[x], acc.at[est.at[jj, 1]],
                                 ssem.at[x], add=True)

            def scale_rows(x, jj):
                @pl.loop(0, _G // 16)
                def _(gg):
                    wv = plsc.bitcast(
                        est[jj, 2, pl.ds(gg * 16, 16)], f32)
                    for t in range(16):
                        r = gg * 16 + t
                        if pairs_mode:
                            rb[x][r, pl.ds(0, 16)] = e0f * wv[t] + e1f
                        else:
                            ws = wv[t]
                            rb[x][r, pl.ds(0, 16)] = (
                                rb[x][r, pl.ds(0, 16)] * ws)
                            rb[x][r, pl.ds(16, 16)] = (
                                rb[x][r, pl.ds(16, 16)] * ws)

            @pl.loop(0, _G)
            def _(r):
                rbA[r, pl.ds(0, 16)] = zeros16
                rbA[r, pl.ds(16, 16)] = zeros16
                if pairs_mode:
                    rbB[r, pl.ds(16, 16)] = zeros16

            zr = n_rows // _NTILES
            zf, zrem = divmod(zr, _G)
            for i in range(zf):
                pltpu.sync_copy(rbA, acc.at[pl.ds(tid * zr + i * _G, _G)])
            if zrem:
                pltpu.sync_copy(rbA.at[pl.ds(0, zrem)],
                                acc.at[pl.ds(tid * zr + zf * _G, zrem)])
            plsc.subcore_barrier()

            @pl.loop(0, nsuper)
            def _(s):
                @pl.when(s > 0)
                def _():
                    drain_scatters(1)
                pltpu.sync_copy(e3ref.at[pl.ds(srow + s * _SSB, _SSB)], est)
                if not pairs_mode:
                    fire_gathers(0, 0)

                @pl.loop(0, _SSB, step=2)
                def _(j):
                    @pl.when(j > 0)
                    def _():
                        drain_scatters(1)
                    if not pairs_mode:
                        fire_gathers(1, j + 1)
                        drain_gathers(0)
                    scale_rows(0, j)
                    fire_scatters(0, j)
                    if not pairs_mode:
                        drain_gathers(1)
                    scale_rows(1, j + 1)
                    fire_scatters(1, j + 1)
                    drain_scatters(0)
                    if not pairs_mode:
                        @pl.when(j < _SSB - 2)
                        def _():
                            fire_gathers(0, j + 2)

            drain_scatters(1)
            plsc.subcore_barrier()

            rpt = n_rows // _NTILES
            cf, crem = divmod(rpt, _G)
            for i in range(cf):
                o = tid * rpt + i * _G
                pltpu.sync_copy(acc.at[pl.ds(o, _G)], out_ref.at[pl.ds(o, _G)])
            if crem:
                o = tid * rpt + cf * _G
                pltpu.sync_copy(acc.at[pl.ds(o, crem)],
                                out_ref.at[pl.ds(o, crem)])
            plsc.subcore_barrier()

        gcs = [g0, g1, g2, g3]

        @pl.when(core == 0)
        def _():
            for c in range(_NCH):
                one_pass(gcs[c], e3w_h, [Sw0, Sw1, Sw2, Sw3][c], _NWP, False)
            one_pass(gcs[0], e3w_h, Pw, _NWP, True)

        @pl.when(core == 1)
        def _():
            for c in range(_NCH):
                one_pass(gcs[c], e3r_h, [Sr0, Sr1, Sr2, Sr3][c], _NRP, False)
            one_pass(gcs[0], e3r_h, Pr, _NRP, True)

    return agg(*gc, e3w, e3r)


_TBLK = 512


def _etype_specs(base):
    def mk(b):
        return pl.BlockSpec((_TBLK, _CC), lambda i, b=b: (i + b, 0))
    return ([mk(base) for _ in range(5)]
            + [pl.BlockSpec((_D, _D), lambda i: (0, 0)),
               pl.BlockSpec((1, _D), lambda i: (0, 0))])


def _mean_part(srefs, p_ref, wt_ref, b_ref):
    acc = jnp.dot(srefs[0][...], wt_ref[pl.ds(0, _CC), :],
                  preferred_element_type=jnp.float32)
    for c in range(1, _NCH):
        acc += jnp.dot(srefs[c][...], wt_ref[pl.ds(c * _CC, _CC), :],
                       preferred_element_type=jnp.float32)
    sw = p_ref[:, 0:1]
    cnt = p_ref[:, 1:2]
    return (acc + sw * b_ref[...]) / jnp.maximum(cnt, 1.0)


def _tc_body1(s0, s1, s2, s3, p, wt, b, o_ref):
    o_ref[...] = _mean_part([s0, s1, s2, s3], p, wt, b)


def _tc_body2(a0, a1, a2, a3, ap, awt, ab, b0, b1, b2, b3, bp, bwt, bb, o_ref):
    o_ref[...] = (_mean_part([a0, a1, a2, a3], ap, awt, ab)
                  + _mean_part([b0, b1, b2, b3], bp, bwt, bb))


def _tc_combine1(agg5, Wt, b, n_out):
    return pl.pallas_call(
        _tc_body1,
        grid=(n_out // _TBLK,),
        in_specs=_etype_specs(0),
        out_specs=pl.BlockSpec((_TBLK, _D), lambda i: (i, 0)),
        out_shape=jax.ShapeDtypeStruct((n_out, _D), jnp.float32),
    )(*agg5, Wt, b)


def _tc_combine2(agg5, base_a, Wta, ba, base_b, Wtb, bb, n_out):
    return pl.pallas_call(
        _tc_body2,
        grid=(n_out // _TBLK,),
        in_specs=_etype_specs(base_a) + _etype_specs(base_b),
        out_specs=pl.BlockSpec((_TBLK, _D), lambda i: (i, 0)),
        out_shape=jax.ShapeDtypeStruct((n_out, _D), jnp.float32),
    )(*agg5, Wta, ba, *agg5, Wtb, bb)


def kernel(feat_word, feat_topic, ei_ww, ei_wt, ei_wd, ei_td, ei_tt,
           w_ww, w_wt, w_wd, w_td, w_tt,
           W_ww, b_ww, W_wt, b_wt, W_wd, b_wd, W_td, b_td, W_tt, b_tt):
    e3w = _pack_edges(ei_ww[0], ei_ww[1], w_ww, _NW, _NWP - _NW)
    src_r = jnp.concatenate([ei_wt[0], ei_tt[0] + _NW, ei_wd[0],
                             ei_td[0] + _NW])
    dst_r = jnp.concatenate([ei_wt[1], ei_tt[1] + _TT0, ei_wd[1] + _WD0,
                             ei_td[1] + _TD0])
    w_r = jnp.concatenate([w_wt, w_tt, w_wd, w_td])
    e3r = _pack_edges(src_r, dst_r, w_r, _TD0 + _ND, _NRP - (_TD0 + _ND))
    gc = [jnp.concatenate([feat_word[:, c * _CC:(c + 1) * _CC],
                           feat_topic[:, c * _CC:(c + 1) * _CC]])
          for c in range(_NCH)]

    outs = _sc_aggregate(gc, e3w, e3r)
    agg_w, agg_r = outs[:5], outs[5:]

    h_word = _tc_combine1(agg_w, W_ww.T, b_ww.reshape(1, _D), _NWP)
    h_topic = _tc_combine2(agg_r, 0, W_wt.T, b_wt.reshape(1, _D),
                           _TT0 // _TBLK, W_tt.T, b_tt.reshape(1, _D), _NTP)
    h_doc = _tc_combine2(agg_r, _WD0 // _TBLK, W_wd.T, b_wd.reshape(1, _D),
                         _TD0 // _TBLK, W_td.T, b_td.reshape(1, _D), _NDP)
    return (h_word[:_NW], h_topic[:_NT], h_doc[:_ND])


# speedup vs baseline: 1.3854x; 1.3854x over previous
import dataclasses
import functools

import jax
import jax.numpy as jnp
from jax import lax
from jax.experimental import pallas as pl
from jax.experimental.pallas import tpu as pltpu
from jax.experimental.pallas import tpu_sc as plsc

_NW, _NT, _ND = 50000, 5000, 10000
_D = 128
_CC = 32
_NCH = _D // _CC
_B = 256
_G = 256
_SSB = 4
_NTILES = 16

_NWP = 50176
_TT0, _WD0, _TD0 = 5120, 10240, 20480
_NRP = 30720
_NTP = 5120
_NDP = 10240

_EMULT = _NTILES * _B * _SSB


def _pack_edges(src, dst, w, trim_lo, trim_n):
    ne = src.shape[0]
    ne_pad = ((ne + _EMULT - 1) // _EMULT) * _EMULT
    pad = ne_pad - ne
    pdst = trim_lo + jnp.arange(pad, dtype=jnp.int32) % trim_n
    wh = lax.bitcast_convert_type(w.astype(jnp.bfloat16),
                                  jnp.uint16).astype(jnp.int32)
    wb = wh | (wh << 16)
    src = jnp.concatenate([src, jnp.zeros((pad,), jnp.int32)])
    dst = jnp.concatenate([dst, pdst])
    wb = jnp.concatenate([wb, jnp.zeros((pad,), jnp.int32)])
    return jnp.stack([src.reshape(-1, _B), dst.reshape(-1, _B),
                      wb.reshape(-1, _B)], axis=1)


def _sc_aggregate(gc, e3w, e3r):
    mesh = plsc.VectorSubcoreMesh(core_axis_name="c", subcore_axis_name="s")
    f32 = jnp.float32
    bf16 = jnp.bfloat16
    out_type = ([jax.ShapeDtypeStruct((_NWP, _CC), bf16) for _ in range(5)]
                + [jax.ShapeDtypeStruct((_NRP, _CC), bf16) for _ in range(5)])
    scratch_types = [
        pltpu.VMEM((_G, _CC), bf16),
        pltpu.VMEM((_G, _CC), bf16),
        pltpu.VMEM((_SSB, 3, _B), jnp.int32),
        pltpu.VMEM_SHARED((_NWP, _CC), bf16),
        pltpu.SemaphoreType.DMA((2,)),
        pltpu.SemaphoreType.DMA((2,)),
    ]
    cparams = pltpu.CompilerParams(use_tc_tiling_on_sc=False)
    if "needs_layout_passes" in pltpu.CompilerParams.__dataclass_fields__:
        cparams = dataclasses.replace(cparams, needs_layout_passes=False)

    @functools.partial(pl.kernel, out_type=out_type, mesh=mesh,
                       scratch_types=scratch_types,
                       compiler_params=cparams)
    def agg(g0, g1, g2, g3, e3w_h, e3r_h,
            Sw0, Sw1, Sw2, Sw3, Pw, Sr0, Sr1, Sr2, Sr3, Pr,
            rbA, rbB, est, acc, gsem, ssem):
        core = lax.axis_index("c")
        tid = lax.axis_index("s")
        rb = [rbA, rbB]

        lane = lax.broadcasted_iota(jnp.int32, (16,), 0)
        hot = jnp.where(lane == 0, 1.0, 0.0)
        z16 = jnp.zeros((16,), f32)
        e0f = plsc.pack(hot, z16, format=plsc.PackFormat.INTERLEAVED)
        e1f = plsc.pack(z16, hot, format=plsc.PackFormat.INTERLEAVED)
        zrow = jnp.zeros((_CC,), bf16)

        def one_pass(G, e3ref, out_ref, n_rows, pairs_mode):
            nblk = e3ref.shape[0] // _NTILES
            nsuper = nblk // _SSB
            srow = tid * nblk

            def drain_scatters(x):
                pltpu.make_async_copy(rb[x], acc.at[pl.ds(0, _G)],
                                      ssem.at[x]).wait()

            def drain_gathers(x):
                pltpu.make_async_copy(G.at[pl.ds(0, _G)], rb[x],
                                      gsem.at[x]).wait()

            def fire_gathers(x, jj):
                pltpu.async_copy(G.at[est.at[jj, 0]], rb[x], gsem.at[x])

            def fire_scatters(x, jj):
                pltpu.async_copy(rb[x], acc.at[est.at[jj, 1]],
                                 ssem.at[x], add=True)

            def scale_rows(x, jj):
                @pl.loop(0, _G // 16)
                def _(gg):
                    wv = est[jj, 2, pl.ds(gg * 16, 16)]
                    for t in range(16):
                        r = gg * 16 + t
                        wbv = plsc.bitcast(
                            jnp.full((16,), 1, jnp.int32) * wv[t], bf16)
                        if pairs_mode:
                            rb[x][r, :] = e0f * wbv + e1f
                        else:
                            rb[x][r, :] = rb[x][r, :] * wbv

            @pl.loop(0, _G)
            def _(r):
                rbA[r, :] = zrow

            zr = n_rows // _NTILES
            zf, zrem = divmod(zr, _G)
            for i in range(zf):
                pltpu.sync_copy(rbA, acc.at[pl.ds(tid * zr + i * _G, _G)])
            if zrem:
                pltpu.sync_copy(rbA.at[pl.ds(0, zrem)],
                                acc.at[pl.ds(tid * zr + zf * _G, zrem)])
            plsc.subcore_barrier()

            @pl.loop(0, nsuper)
            def _(s):
                @pl.when(s > 0)
                def _():
                    drain_scatters(1)
                pltpu.sync_copy(e3ref.at[pl.ds(srow + s * _SSB, _SSB)], est)
                if not pairs_mode:
                    fire_gathers(0, 0)

                @pl.loop(0, _SSB, step=2)
                def _(j):
                    @pl.when(j > 0)
                    def _():
                        drain_scatters(1)
                    if not pairs_mode:
                        fire_gathers(1, j + 1)
                        drain_gathers(0)
                    scale_rows(0, j)
                    fire_scatters(0, j)
                    if not pairs_mode:
                        drain_gathers(1)
                    scale_rows(1, j + 1)
                    fire_scatters(1, j + 1)
                    drain_scatters(0)
                    if not pairs_mode:
                        @pl.when(j < _SSB - 2)
                        def _():
                            fire_gathers(0, j + 2)

            drain_scatters(1)
            plsc.subcore_barrier()

            rpt = n_rows // _NTILES
            cf, crem = divmod(rpt, _G)
            for i in range(cf):
                o = tid * rpt + i * _G
                pltpu.sync_copy(acc.at[pl.ds(o, _G)], out_ref.at[pl.ds(o, _G)])
            if crem:
                o = tid * rpt + cf * _G
                pltpu.sync_copy(acc.at[pl.ds(o, crem)],
                                out_ref.at[pl.ds(o, crem)])
            plsc.subcore_barrier()

        gcs = [g0, g1, g2, g3]

        @pl.when(core == 0)
        def _():
            for c in range(_NCH):
                one_pass(gcs[c], e3w_h, [Sw0, Sw1, Sw2, Sw3][c], _NWP, False)
            one_pass(gcs[0], e3w_h, Pw, _NWP, True)

        @pl.when(core == 1)
        def _():
            for c in range(_NCH):
                one_pass(gcs[c], e3r_h, [Sr0, Sr1, Sr2, Sr3][c], _NRP, False)
            one_pass(gcs[0], e3r_h, Pr, _NRP, True)

    return agg(*gc, e3w, e3r)


_TBLK = 512


def _etype_specs(base):
    def mk(b):
        return pl.BlockSpec((_TBLK, _CC), lambda i, b=b: (i + b, 0))
    return ([mk(base) for _ in range(5)]
            + [pl.BlockSpec((_D, _D), lambda i: (0, 0)),
               pl.BlockSpec((1, _D), lambda i: (0, 0))])


def _mean_part(srefs, p_ref, wt_ref, b_ref):
    acc = jnp.dot(srefs[0][...], wt_ref[pl.ds(0, _CC), :],
                  preferred_element_type=jnp.float32)
    for c in range(1, _NCH):
        acc += jnp.dot(srefs[c][...], wt_ref[pl.ds(c * _CC, _CC), :],
                       preferred_element_type=jnp.float32)
    sw = p_ref[:, 0:1].astype(jnp.float32)
    cnt = p_ref[:, 1:2].astype(jnp.float32)
    return (acc + sw * b_ref[...]) / jnp.maximum(cnt, 1.0)


def _tc_body1(s0, s1, s2, s3, p, wt, b, o_ref):
    o_ref[...] = _mean_part([s0, s1, s2, s3], p, wt, b)


def _tc_body2(a0, a1, a2, a3, ap, awt, ab, b0, b1, b2, b3, bp, bwt, bb, o_ref):
    o_ref[...] = (_mean_part([a0, a1, a2, a3], ap, awt, ab)
                  + _mean_part([b0, b1, b2, b3], bp, bwt, bb))


def _tc_combine1(agg5, Wt, b, n_out):
    return pl.pallas_call(
        _tc_body1,
        grid=(n_out // _TBLK,),
        in_specs=_etype_specs(0),
        out_specs=pl.BlockSpec((_TBLK, _D), lambda i: (i, 0)),
        out_shape=jax.ShapeDtypeStruct((n_out, _D), jnp.float32),
    )(*agg5, Wt, b)


def _tc_combine2(agg5, base_a, Wta, ba, base_b, Wtb, bb, n_out):
    return pl.pallas_call(
        _tc_body2,
        grid=(n_out // _TBLK,),
        in_specs=_etype_specs(base_a) + _etype_specs(base_b),
        out_specs=pl.BlockSpec((_TBLK, _D), lambda i: (i, 0)),
        out_shape=jax.ShapeDtypeStruct((n_out, _D), jnp.float32),
    )(*agg5, Wta, ba, *agg5, Wtb, bb)


def kernel(feat_word, feat_topic, ei_ww, ei_wt, ei_wd, ei_td, ei_tt,
           w_ww, w_wt, w_wd, w_td, w_tt,
           W_ww, b_ww, W_wt, b_wt, W_wd, b_wd, W_td, b_td, W_tt, b_tt):
    e3w = _pack_edges(ei_ww[0], ei_ww[1], w_ww, _NW, _NWP - _NW)
    src_r = jnp.concatenate([ei_wt[0], ei_tt[0] + _NW, ei_wd[0],
                             ei_td[0] + _NW])
    dst_r = jnp.concatenate([ei_wt[1], ei_tt[1] + _TT0, ei_wd[1] + _WD0,
                             ei_td[1] + _TD0])
    w_r = jnp.concatenate([w_wt, w_tt, w_wd, w_td])
    e3r = _pack_edges(src_r, dst_r, w_r, _TD0 + _ND, _NRP - (_TD0 + _ND))
    gc = [jnp.concatenate([feat_word[:, c * _CC:(c + 1) * _CC],
                           feat_topic[:, c * _CC:(c + 1) * _CC]]
                          ).astype(jnp.bfloat16)
          for c in range(_NCH)]

    outs = _sc_aggregate(gc, e3w, e3r)
    agg_w, agg_r = outs[:5], outs[5:]

    bf = jnp.bfloat16
    h_word = _tc_combine1(agg_w, W_ww.T.astype(bf), b_ww.reshape(1, _D), _NWP)
    h_topic = _tc_combine2(agg_r, 0, W_wt.T.astype(bf), b_wt.reshape(1, _D),
                           _TT0 // _TBLK, W_tt.T.astype(bf),
                           b_tt.reshape(1, _D), _NTP)
    h_doc = _tc_combine2(agg_r, _WD0 // _TBLK, W_wd.T.astype(bf),
                         b_wd.reshape(1, _D), _TD0 // _TBLK,
                         W_td.T.astype(bf), b_td.reshape(1, _D), _NDP)
    return (h_word[:_NW], h_topic[:_NT], h_doc[:_ND])
